# baseline (speedup 1.0000x reference)
import jax
import jax.numpy as jnp
from jax import lax
from jax.experimental import pallas as pl
from jax.experimental.pallas import tpu as pltpu


def _stats_kernel(x2c_ref, x1c_ref, W2_ref, W1_ref, g2_ref, be2_ref,
                  g1_ref, be1_ref, scale2_ref, shift2_ref, scale1_ref,
                  shift1_ref, G2_ref, s2_ref, G1_ref, s1_ref, *, nsteps,
                  M2, M1):
    f32 = jnp.float32
    i = pl.program_id(0)

    @pl.when(i == 0)
    def _init():
        G2_ref[...] = jnp.zeros_like(G2_ref)
        s2_ref[...] = jnp.zeros_like(s2_ref)
        G1_ref[...] = jnp.zeros_like(G1_ref)
        s1_ref[...] = jnp.zeros_like(s1_ref)

    x2 = x2c_ref[...]
    x1 = x1c_ref[...]
    G2_ref[...] += lax.dot_general(x2, x2, (((0,), (0,)), ((), ())),
                                   preferred_element_type=f32)
    s2_ref[...] += jnp.sum(x2, axis=0, keepdims=True)
    G1_ref[...] += lax.dot_general(x1, x1, (((0,), (0,)), ((), ())),
                                   preferred_element_type=f32)
    s1_ref[...] += jnp.sum(x1, axis=0, keepdims=True)

    @pl.when(i == nsteps - 1)
    def _fold():
        def bn_fold(G, s, W, g, be, M):
            C = G.shape[0]
            xbar = s * (1.0 / M)
            proj = lax.dot_general(xbar, W, (((1,), (1,)), ((), ())),
                                   preferred_element_type=f32)
            A = lax.dot_general(W, G, (((1,), (0,)), ((), ())),
                                preferred_element_type=f32)
            ones = jnp.ones((1, C), f32)
            ey2 = lax.dot_general(ones, A * W, (((1,), (1,)), ((), ())),
                                  preferred_element_type=f32) * (1.0 / M)
            var = ey2 - proj * proj
            scale = g / jnp.sqrt(var + 1e-5)
            shift = be - proj * scale
            return scale, shift

        sc2, sh2 = bn_fold(G2_ref[...], s2_ref[...], W2_ref[...],
                           g2_ref[...], be2_ref[...], M2)
        sc1, sh1 = bn_fold(G1_ref[...], s1_ref[...], W1_ref[...],
                           g1_ref[...], be1_ref[...], M1)
        scale2_ref[...] = sc2
        shift2_ref[...] = sh2
        scale1_ref[...] = sc1
        shift1_ref[...] = sh1


def _h2_kernel(x2c_ref, W2_ref, scale2_ref, shift2_ref, h2_ref):
    y = lax.dot_general(x2c_ref[...], W2_ref[...], (((1,), (1,)), ((), ())),
                        preferred_element_type=jnp.float32)
    h2_ref[...] = jnp.maximum(y * scale2_ref[...] + shift2_ref[...], 0.0)


def _interp_kernel(p1m2_ref, n1_ref, x1_ref, p2t_ref, n2_ref, h2_ref,
                   W1_ref, scale1_ref, shift1_ref, out_ref):
    f32 = jnp.float32
    p1m2 = p1m2_ref[0]
    p2t = p2t_ref[0]
    n1b = n1_ref[0]
    n2b = n2_ref[0]
    R = p1m2.shape[0]
    N2 = p2t.shape[1]

    inner2 = lax.dot_general(p1m2, p2t, (((1,), (0,)), ((), ())),
                             preferred_element_type=f32)
    d = inner2 + n2b

    wmat = jnp.zeros((R, N2), f32)
    rsum = jnp.zeros((R, 1), f32)
    for k in range(3):
        mk = jnp.min(d, axis=1, keepdims=True)
        sel = d == mk
        if k < 2:
            d = jnp.where(sel, jnp.inf, d)
        r = 1.0 / (jnp.maximum(mk + n1b, 0.0) + 1e-8)
        rsum = rsum + r
        wmat = wmat + jnp.where(sel, r, 0.0)

    h2b = h2_ref[0]
    interp = lax.dot_general(wmat, h2b, (((1,), (0,)), ((), ())),
                             preferred_element_type=f32)
    interp = interp * (1.0 / rsum)

    y1 = lax.dot_general(x1_ref[0], W1_ref[...], (((1,), (1,)), ((), ())),
                         preferred_element_type=f32)
    h1 = jnp.maximum(y1 * scale1_ref[...] + shift1_ref[...], 0.0)
    out_ref[0] = interp + h1


def kernel(p1, x1, p2, x2, W2, b2, g2, be2, W1, b1, g1, be1):
    import functools
    B, N1, _ = p1.shape
    _, N2, C2 = x2.shape
    D = W2.shape[0]
    f32 = jnp.float32

    M2, M1 = B * N2, B * N1
    x2r = x2.reshape(M2, C2)
    x1r = x1.reshape(M1, D)
    g2r, be2r = g2.reshape(1, D), be2.reshape(1, D)
    g1r, be1r = g1.reshape(1, D), be1.reshape(1, D)

    SSTEPS = 8
    c2, c1 = M2 // SSTEPS, M1 // SSTEPS
    vec_spec = pl.BlockSpec((1, D), lambda i: (0, 0))
    scale2, shift2, scale1, shift1 = pl.pallas_call(
        functools.partial(_stats_kernel, nsteps=SSTEPS, M2=M2, M1=M1),
        grid=(SSTEPS,),
        in_specs=[
            pl.BlockSpec((c2, C2), lambda i: (i, 0)),
            pl.BlockSpec((c1, D), lambda i: (i, 0)),
            pl.BlockSpec((D, C2), lambda i: (0, 0)),
            pl.BlockSpec((D, D), lambda i: (0, 0)),
            vec_spec, vec_spec, vec_spec, vec_spec,
        ],
        out_specs=(vec_spec, vec_spec, vec_spec, vec_spec),
        out_shape=(jax.ShapeDtypeStruct((1, D), f32),) * 4,
        scratch_shapes=[
            pltpu.VMEM((C2, C2), f32), pltpu.VMEM((1, C2), f32),
            pltpu.VMEM((D, D), f32), pltpu.VMEM((1, D), f32),
        ],
    )(x2r, x1r, W2, W1, g2r, be2r, g1r, be1r)

    h2r = pl.pallas_call(
        _h2_kernel,
        grid=(SSTEPS,),
        in_specs=[
            pl.BlockSpec((c2, C2), lambda i: (i, 0)),
            pl.BlockSpec((D, C2), lambda i: (0, 0)),
            vec_spec, vec_spec,
        ],
        out_specs=pl.BlockSpec((c2, D), lambda i: (i, 0)),
        out_shape=jax.ShapeDtypeStruct((M2, D), f32),
    )(x2r, W2, scale2, shift2)
    h2 = h2r.reshape(B, N2, D)

    p2t = jnp.transpose(p2, (0, 2, 1))
    p1m2 = -2.0 * p1
    n1 = jnp.sum(p1 * p1, axis=2, keepdims=True)
    n2 = jnp.sum(p2t * p2t, axis=1, keepdims=True)

    R = 512
    grid = (B, N1 // R)
    out = pl.pallas_call(
        _interp_kernel,
        grid=grid,
        in_specs=[
            pl.BlockSpec((1, R, 3), lambda b, i: (b, i, 0)),
            pl.BlockSpec((1, R, 1), lambda b, i: (b, i, 0)),
            pl.BlockSpec((1, R, D), lambda b, i: (b, i, 0)),
            pl.BlockSpec((1, 3, N2), lambda b, i: (b, 0, 0)),
            pl.BlockSpec((1, 1, N2), lambda b, i: (b, 0, 0)),
            pl.BlockSpec((1, N2, D), lambda b, i: (b, 0, 0)),
            pl.BlockSpec((D, D), lambda b, i: (0, 0)),
            pl.BlockSpec((1, D), lambda b, i: (0, 0)),
            pl.BlockSpec((1, D), lambda b, i: (0, 0)),
        ],
        out_specs=pl.BlockSpec((1, R, D), lambda b, i: (b, i, 0)),
        out_shape=jax.ShapeDtypeStruct((B, N1, D), f32),
    )(p1m2, n1, x1, p2t, n2, h2, W1, scale1, shift1)
    return out


# speedup vs baseline: 1.1959x; 1.1959x over previous
import jax
import jax.numpy as jnp
from jax import lax
from jax.experimental import pallas as pl
from jax.experimental.pallas import tpu as pltpu


def _stats_kernel(x2c_ref, x1c_ref, W2_ref, W1_ref, g2_ref, be2_ref,
                  g1_ref, be1_ref, scale2_ref, shift2_ref, scale1_ref,
                  shift1_ref, G2_ref, s2_ref, G1_ref, s1_ref, *, nsteps,
                  M2, M1):
    f32 = jnp.float32
    i = pl.program_id(0)

    @pl.when(i == 0)
    def _init():
        G2_ref[...] = jnp.zeros_like(G2_ref)
        s2_ref[...] = jnp.zeros_like(s2_ref)
        G1_ref[...] = jnp.zeros_like(G1_ref)
        s1_ref[...] = jnp.zeros_like(s1_ref)

    x2 = x2c_ref[...]
    x1 = x1c_ref[...]
    G2_ref[...] += lax.dot_general(x2, x2, (((0,), (0,)), ((), ())),
                                   preferred_element_type=f32)
    s2_ref[...] += jnp.sum(x2, axis=0, keepdims=True)
    G1_ref[...] += lax.dot_general(x1, x1, (((0,), (0,)), ((), ())),
                                   preferred_element_type=f32)
    s1_ref[...] += jnp.sum(x1, axis=0, keepdims=True)

    @pl.when(i == nsteps - 1)
    def _fold():
        def bn_fold(G, s, W, g, be, M):
            C = G.shape[0]
            xbar = s * (1.0 / M)
            proj = lax.dot_general(xbar, W, (((1,), (1,)), ((), ())),
                                   preferred_element_type=f32)
            A = lax.dot_general(W, G, (((1,), (0,)), ((), ())),
                                preferred_element_type=f32)
            ones = jnp.ones((1, C), f32)
            ey2 = lax.dot_general(ones, A * W, (((1,), (1,)), ((), ())),
                                  preferred_element_type=f32) * (1.0 / M)
            var = ey2 - proj * proj
            scale = g / jnp.sqrt(var + 1e-5)
            shift = be - proj * scale
            return scale, shift

        sc2, sh2 = bn_fold(G2_ref[...], s2_ref[...], W2_ref[...],
                           g2_ref[...], be2_ref[...], M2)
        sc1, sh1 = bn_fold(G1_ref[...], s1_ref[...], W1_ref[...],
                           g1_ref[...], be1_ref[...], M1)
        scale2_ref[...] = sc2
        shift2_ref[...] = sh2
        scale1_ref[...] = sc1
        shift1_ref[...] = sh1


def _h2_kernel(x2c_ref, W2_ref, scale2_ref, shift2_ref, h2_ref):
    y = lax.dot_general(x2c_ref[...], W2_ref[...], (((1,), (1,)), ((), ())),
                        preferred_element_type=jnp.float32)
    h2_ref[...] = jnp.maximum(y * scale2_ref[...] + shift2_ref[...], 0.0)


def _interp_kernel(p1_ref, x1_ref, p2t_ref, h2_ref,
                   W1_ref, scale1_ref, shift1_ref, out_ref):
    f32 = jnp.float32
    p1b = p1_ref[0]
    p2t = p2t_ref[0]
    R = p1b.shape[0]
    N2 = p2t.shape[1]
    n1b = jnp.sum(p1b * p1b, axis=1, keepdims=True)
    n2b = jnp.sum(p2t * p2t, axis=0, keepdims=True)

    inner2 = lax.dot_general(-2.0 * p1b, p2t, (((1,), (0,)), ((), ())),
                             preferred_element_type=f32)
    d = inner2 + n2b

    wmat = jnp.zeros((R, N2), f32)
    rsum = jnp.zeros((R, 1), f32)
    for k in range(3):
        mk = jnp.min(d, axis=1, keepdims=True)
        sel = d == mk
        if k < 2:
            d = jnp.where(sel, jnp.inf, d)
        r = 1.0 / (jnp.maximum(mk + n1b, 0.0) + 1e-8)
        rsum = rsum + r
        wmat = wmat + jnp.where(sel, r, 0.0)

    h2b = h2_ref[0]
    interp = lax.dot_general(wmat, h2b, (((1,), (0,)), ((), ())),
                             preferred_element_type=f32)
    interp = interp * (1.0 / rsum)

    y1 = lax.dot_general(x1_ref[0], W1_ref[...], (((1,), (1,)), ((), ())),
                         preferred_element_type=f32)
    h1 = jnp.maximum(y1 * scale1_ref[...] + shift1_ref[...], 0.0)
    out_ref[0] = interp + h1


def kernel(p1, x1, p2, x2, W2, b2, g2, be2, W1, b1, g1, be1):
    import functools
    B, N1, _ = p1.shape
    _, N2, C2 = x2.shape
    D = W2.shape[0]
    f32 = jnp.float32

    M2, M1 = B * N2, B * N1
    x2r = x2.reshape(M2, C2)
    x1r = x1.reshape(M1, D)
    g2r, be2r = g2.reshape(1, D), be2.reshape(1, D)
    g1r, be1r = g1.reshape(1, D), be1.reshape(1, D)

    SSTEPS = 8
    c2, c1 = M2 // SSTEPS, M1 // SSTEPS
    vec_spec = pl.BlockSpec((1, D), lambda i: (0, 0))
    scale2, shift2, scale1, shift1 = pl.pallas_call(
        functools.partial(_stats_kernel, nsteps=SSTEPS, M2=M2, M1=M1),
        grid=(SSTEPS,),
        in_specs=[
            pl.BlockSpec((c2, C2), lambda i: (i, 0)),
            pl.BlockSpec((c1, D), lambda i: (i, 0)),
            pl.BlockSpec((D, C2), lambda i: (0, 0)),
            pl.BlockSpec((D, D), lambda i: (0, 0)),
            vec_spec, vec_spec, vec_spec, vec_spec,
        ],
        out_specs=(vec_spec, vec_spec, vec_spec, vec_spec),
        out_shape=(jax.ShapeDtypeStruct((1, D), f32),) * 4,
        scratch_shapes=[
            pltpu.VMEM((C2, C2), f32), pltpu.VMEM((1, C2), f32),
            pltpu.VMEM((D, D), f32), pltpu.VMEM((1, D), f32),
        ],
    )(x2r, x1r, W2, W1, g2r, be2r, g1r, be1r)

    h2r = pl.pallas_call(
        _h2_kernel,
        grid=(SSTEPS,),
        in_specs=[
            pl.BlockSpec((c2, C2), lambda i: (i, 0)),
            pl.BlockSpec((D, C2), lambda i: (0, 0)),
            vec_spec, vec_spec,
        ],
        out_specs=pl.BlockSpec((c2, D), lambda i: (i, 0)),
        out_shape=jax.ShapeDtypeStruct((M2, D), f32),
    )(x2r, W2, scale2, shift2)
    h2 = h2r.reshape(B, N2, D)

    p2t = jnp.transpose(p2, (0, 2, 1))

    R = 512
    grid = (B, N1 // R)
    out = pl.pallas_call(
        _interp_kernel,
        grid=grid,
        in_specs=[
            pl.BlockSpec((1, R, 3), lambda b, i: (b, i, 0)),
            pl.BlockSpec((1, R, D), lambda b, i: (b, i, 0)),
            pl.BlockSpec((1, 3, N2), lambda b, i: (b, 0, 0)),
            pl.BlockSpec((1, N2, D), lambda b, i: (b, 0, 0)),
            pl.BlockSpec((D, D), lambda b, i: (0, 0)),
            pl.BlockSpec((1, D), lambda b, i: (0, 0)),
            pl.BlockSpec((1, D), lambda b, i: (0, 0)),
        ],
        out_specs=pl.BlockSpec((1, R, D), lambda b, i: (b, i, 0)),
        out_shape=jax.ShapeDtypeStruct((B, N1, D), f32),
    )(p1, x1, p2t, h2, W1, scale1, shift1)
    return out
